# baseline (speedup 1.0000x reference)
import functools

import jax
import jax.numpy as jnp
import numpy as np
from jax import lax
from jax.experimental import pallas as pl
from jax.experimental.pallas import tpu as pltpu
from jax.experimental.pallas import tpu_sc as plsc

N_NODES = 50000
DIM = 64
HIDDEN = 128
N_REL = 4
E_PER_REL = 200000
EPS = 1e-5

NC = 2
NS = 16
LANES = 16
CW = 32
K = 128
EPT = 13312
NB = EPT // K
IB = 26
E_PAD = NS * EPT
TRASH = N_NODES
ACC_ROWS = 50176
RPT = ACC_ROWS // NS
ZROWS = RPT // 8


def _sc_agg_kernel(nchunk, with_counts):
  mesh = plsc.VectorSubcoreMesh(core_axis_name="c", subcore_axis_name="s")
  out_type = [jax.ShapeDtypeStruct((N_REL, N_NODES, CW * nchunk), jnp.float32)]
  if with_counts:
    out_type.append(jax.ShapeDtypeStruct((N_REL, N_NODES, 8), jnp.float32))

  scratch = (
      [pltpu.VMEM((IB, K), jnp.int32)] * 2
      + [pltpu.VMEM((K, CW), jnp.float32)] * 2
      + [pltpu.VMEM((ZROWS, CW), jnp.float32)]
      + [pltpu.VMEM_SHARED((ACC_ROWS, CW), jnp.float32)]
      + [pltpu.SemaphoreType.DMA] * 2
  )

  def body(*refs):
    srcs, dsts = refs[0], refs[1]
    tables = refs[2:2 + nchunk]
    out = refs[2 + nchunk]
    cnt_out = refs[3 + nchunk] if with_counts else None
    src_v, dst_v, rows0, rows1, zbuf, acc, sem0, sem1 = refs[-8:]

    c = lax.axis_index("c")
    s = lax.axis_index("s")
    n0 = s * RPT

    zero16 = jnp.zeros((LANES,), jnp.float32)

    def fill_z(i, carry):
      zbuf[i, pl.ds(0, LANES)] = zero16
      zbuf[i, pl.ds(LANES, LANES)] = zero16
      return carry
    lax.fori_loop(0, ZROWS, fill_z, 0)

    def zero_acc():
      for j in range(RPT // ZROWS):
        pltpu.sync_copy(zbuf, acc.at[pl.ds(n0 + j * ZROWS, ZROWS)])

    def dump(r, col):
      @pl.when(s < NS - 1)
      def _():
        pltpu.sync_copy(acc.at[pl.ds(n0, RPT)],
                        out.at[r, pl.ds(n0, RPT), pl.ds(col, CW)])
      @pl.when(s == NS - 1)
      def _():
        last = (NS - 1) * RPT
        pltpu.sync_copy(acc.at[pl.ds(last, N_NODES - last)],
                        out.at[r, pl.ds(last, N_NODES - last), pl.ds(col, CW)])

    for q in range(nchunk):
      @pl.when(q % NC == c)
      def _(q=q):
        table = tables[q]

        def gather(jj, buf, sem):
          pltpu.async_copy(table.at[src_v.at[jj]], buf, sem)

        def gwait(buf, sem):
          pltpu.make_async_copy(table.at[src_v.at[0]], buf, sem).wait()

        def rel_body(r, carry0):
          zero_acc()
          plsc.subcore_barrier()

          def chunk(kk, carry):
            pltpu.sync_copy(srcs.at[r, s, pl.ds(kk * IB, IB)], src_v)
            pltpu.sync_copy(dsts.at[r, s, pl.ds(kk * IB, IB)], dst_v)
            gather(0, rows0, sem0)
            gather(1, rows1, sem1)

            def pair(j, carry2):
              b0 = 2 * j
              gwait(rows0, sem0)
              pltpu.sync_copy(rows0, acc.at[dst_v.at[b0]], add=True)
              gather(jnp.minimum(b0 + 2, IB - 1), rows0, sem0)
              gwait(rows1, sem1)
              pltpu.sync_copy(rows1, acc.at[dst_v.at[b0 + 1]], add=True)
              gather(jnp.minimum(b0 + 3, IB - 1), rows1, sem1)
              return carry2
            lax.fori_loop(0, IB // 2, pair, 0)
            gwait(rows0, sem0)
            gwait(rows1, sem1)
            return carry
          lax.fori_loop(0, NB // IB, chunk, 0)
          plsc.subcore_barrier()
          dump(r, CW * q)
          plsc.subcore_barrier()
          return carry0
        lax.fori_loop(0, N_REL, rel_body, 0)

    if with_counts:
      one16 = jnp.ones((LANES,), jnp.float32)

      def fill_o(i, carry):
        rows0[i, pl.ds(0, LANES)] = one16
        rows0[i, pl.ds(LANES, LANES)] = one16
        return carry
      lax.fori_loop(0, K, fill_o, 0)

      def cnt_body(r, carry0):
        @pl.when(r % NC == c)
        def _():
          zero_acc()
          plsc.subcore_barrier()

          def chunk(kk, carry):
            pltpu.sync_copy(dsts.at[r, s, pl.ds(kk * IB, IB)], dst_v)

            def fire(b, carry2):
              pltpu.async_copy(rows0, acc.at[dst_v.at[b]], sem1, add=True)
              return carry2
            lax.fori_loop(0, IB, fire, 0)

            def drain(b, carry2):
              pltpu.make_async_copy(rows0, acc.at[dst_v.at[0]], sem1).wait()
              return carry2
            lax.fori_loop(0, IB, drain, 0)
            return carry
          lax.fori_loop(0, NB // IB, chunk, 0)
          plsc.subcore_barrier()

          @pl.when(s < NS - 1)
          def _():
            pltpu.sync_copy(acc.at[pl.ds(n0, RPT), pl.ds(0, 8)],
                            cnt_out.at[r, pl.ds(n0, RPT)])
          @pl.when(s == NS - 1)
          def _():
            last = (NS - 1) * RPT
            pltpu.sync_copy(acc.at[pl.ds(last, N_NODES - last), pl.ds(0, 8)],
                            cnt_out.at[r, pl.ds(last, N_NODES - last)])
          plsc.subcore_barrier()
        return carry0
      lax.fori_loop(0, N_REL, cnt_body, 0)

  return pl.kernel(body, out_type=out_type, mesh=mesh, scratch_types=scratch,
                   compiler_params=pltpu.CompilerParams(
                       use_tc_tiling_on_sc=False))


_BLK = 2000
_GRID = N_NODES // _BLK
_AROWS1 = _BLK * DIM // 128


def _tc_layer1(emb, a1p, cnt, w_root, w_rel_s, b1, gamma, beta):
  def body(emb_b, a_b, cnt_b, wr_b, ws_b, b1_b, g_b, be_b, h_o):
    x = emb_b[...]
    a = a_b[...]
    cnt = cnt_b[:, :, 0]
    parts = []
    for r in range(N_REL):
      d = jnp.clip(cnt[r], 1.0, None)[:, None]
      parts.append(a[r] / d)
    mean_all = jnp.concatenate(parts, axis=1)
    acc = (jnp.dot(x, wr_b[...], preferred_element_type=jnp.float32)
           + jnp.dot(mean_all, ws_b[...], preferred_element_type=jnp.float32)
           + b1_b[...])
    h = acc * (g_b[...] / np.sqrt(1.0 + EPS)) + be_b[...]
    h_o[...] = jnp.where(h > 0, h, jnp.exp(jnp.minimum(h, 0.0)) - 1.0)

  blk_n = lambda i: (i, 0)
  blk_rel = lambda i: (0, i, 0)
  blk_w = lambda i: (0, 0)
  return pl.pallas_call(
      body,
      grid=(_GRID,),
      in_specs=[
          pl.BlockSpec((_BLK, DIM), blk_n),
          pl.BlockSpec((N_REL, _BLK, DIM), blk_rel),
          pl.BlockSpec((N_REL, _BLK, 8), blk_rel),
          pl.BlockSpec((DIM, HIDDEN), blk_w),
          pl.BlockSpec((N_REL * DIM, HIDDEN), blk_w),
          pl.BlockSpec((1, HIDDEN), blk_w),
          pl.BlockSpec((1, HIDDEN), blk_w),
          pl.BlockSpec((1, HIDDEN), blk_w),
      ],
      out_specs=pl.BlockSpec((_BLK, HIDDEN), blk_n),
      out_shape=jax.ShapeDtypeStruct((N_NODES, HIDDEN), jnp.float32),
  )(emb, a1p, cnt, w_root, w_rel_s, b1, gamma, beta)


def _tc_layer2(h, a2, cnt, w_root, w_rel_s, b2):
  def body(h_b, a_b, cnt_b, wr_b, ws_b, b2_b, z_o):
    x = h_b[...]
    cnt = cnt_b[:, :, 0]
    parts = []
    for r in range(N_REL):
      d = jnp.clip(cnt[r], 1.0, None)[:, None]
      parts.append(a_b[r] / d)
    mean_all = jnp.concatenate(parts, axis=1)
    z_o[...] = (jnp.dot(x, wr_b[...], preferred_element_type=jnp.float32)
                + jnp.dot(mean_all, ws_b[...], preferred_element_type=jnp.float32)
                + b2_b[...])

  blk_n = lambda i: (i, 0)
  blk_rel = lambda i: (0, i, 0)
  blk_w = lambda i: (0, 0)
  return pl.pallas_call(
      body,
      grid=(_GRID,),
      in_specs=[
          pl.BlockSpec((_BLK, HIDDEN), blk_n),
          pl.BlockSpec((N_REL, _BLK, HIDDEN), blk_rel),
          pl.BlockSpec((N_REL, _BLK, 8), blk_rel),
          pl.BlockSpec((HIDDEN, DIM), blk_w),
          pl.BlockSpec((N_REL * HIDDEN, DIM), blk_w),
          pl.BlockSpec((1, DIM), blk_w),
      ],
      out_specs=pl.BlockSpec((_BLK, DIM), blk_n),
      out_shape=jax.ShapeDtypeStruct((N_NODES, DIM), jnp.float32),
  )(h, a2, cnt, w_root, w_rel_s, b2)


@jax.jit
def kernel(rel_edges_visible, emb, W_rel1, W_root1, b1, bn_gamma, bn_beta,
           W_rel2, W_root2, b2):
  edges = rel_edges_visible.astype(jnp.int32)
  src = edges[:, 0, :]
  dst = edges[:, 1, :]
  pad = E_PAD - E_PER_REL
  src_p = jnp.concatenate(
      [src, jnp.zeros((N_REL, pad), jnp.int32)], axis=1
  ).reshape(N_REL, NS, NB, K)
  dst_p = jnp.concatenate(
      [dst, jnp.full((N_REL, pad), TRASH, jnp.int32)], axis=1
  ).reshape(N_REL, NS, NB, K)

  emb_c0 = emb[:, 0:32]
  emb_c1 = emb[:, 32:64]

  agg1, cnt = _sc_agg_kernel(2, True)(src_p, dst_p, emb_c0, emb_c1)

  h = _tc_layer1(
      emb, agg1, cnt,
      W_root1, W_rel1.reshape(N_REL * DIM, HIDDEN),
      b1.reshape(1, HIDDEN), bn_gamma.reshape(1, HIDDEN),
      bn_beta.reshape(1, HIDDEN))

  h_tabs = [h[:, CW * q:CW * (q + 1)] for q in range(4)]
  (a2,) = _sc_agg_kernel(4, False)(src_p, dst_p, *h_tabs)

  z = _tc_layer2(h, a2, cnt,
                 W_root2, W_rel2.reshape(N_REL * HIDDEN, DIM),
                 b2.reshape(1, DIM))
  return z


# speedup vs baseline: 1.9154x; 1.9154x over previous
import functools

import jax
import jax.numpy as jnp
import numpy as np
from jax import lax
from jax.experimental import pallas as pl
from jax.experimental.pallas import tpu as pltpu
from jax.experimental.pallas import tpu_sc as plsc

N_NODES = 50000
DIM = 64
HIDDEN = 128
N_REL = 4
E_PER_REL = 200000
EPS = 1e-5

NC = 2
NS = 16
LANES = 16
CW = 32
K = 128
EPT = 12544
NB = EPT // K
IB = 14
E_PAD = NS * EPT
TRASH = N_NODES
ACC_ROWS = 50176
RPT = ACC_ROWS // NS
ZROWS = RPT // 8


def _sc_agg_kernel(nchunk, with_counts):
  mesh = plsc.VectorSubcoreMesh(core_axis_name="c", subcore_axis_name="s")
  out_type = [jax.ShapeDtypeStruct((N_REL, N_NODES, CW * nchunk), jnp.float32)]
  if with_counts:
    out_type.append(jax.ShapeDtypeStruct((N_REL, N_NODES, 8), jnp.float32))

  scratch = (
      [pltpu.VMEM((IB, K), jnp.int32)] * 2
      + [pltpu.VMEM((K, CW), jnp.float32)] * 2
      + [pltpu.VMEM((ZROWS, CW), jnp.float32)]
      + [pltpu.VMEM_SHARED((ACC_ROWS, CW), jnp.float32)]
      + [pltpu.SemaphoreType.DMA] * 2
  )

  def body(*refs):
    srcs, dsts = refs[0], refs[1]
    tables = refs[2:2 + nchunk]
    out = refs[2 + nchunk]
    cnt_out = refs[3 + nchunk] if with_counts else None
    src_v, dst_v, rows0, rows1, zbuf, acc, sem0, sem1 = refs[-8:]

    c = lax.axis_index("c")
    s = lax.axis_index("s")
    n0 = s * RPT

    zero16 = jnp.zeros((LANES,), jnp.float32)

    def fill_z(i, carry):
      zbuf[i, pl.ds(0, LANES)] = zero16
      zbuf[i, pl.ds(LANES, LANES)] = zero16
      return carry
    lax.fori_loop(0, ZROWS, fill_z, 0)

    def zero_acc():
      for j in range(RPT // ZROWS):
        pltpu.sync_copy(zbuf, acc.at[pl.ds(n0 + j * ZROWS, ZROWS)])

    def dump(r, col):
      @pl.when(s < NS - 1)
      def _():
        pltpu.sync_copy(acc.at[pl.ds(n0, RPT)],
                        out.at[r, pl.ds(n0, RPT), pl.ds(col, CW)])
      @pl.when(s == NS - 1)
      def _():
        last = (NS - 1) * RPT
        pltpu.sync_copy(acc.at[pl.ds(last, N_NODES - last)],
                        out.at[r, pl.ds(last, N_NODES - last), pl.ds(col, CW)])

    for q in range(nchunk):
      @pl.when(q % NC == c)
      def _(q=q):
        table = tables[q]

        def gather(jj, buf, sem):
          pltpu.async_copy(table.at[src_v.at[jj]], buf, sem)

        def gwait(buf, sem):
          pltpu.make_async_copy(table.at[src_v.at[0]], buf, sem).wait()

        def rel_body(r, carry0):
          zero_acc()
          plsc.subcore_barrier()

          def chunk(kk, carry):
            pltpu.sync_copy(srcs.at[r, s, pl.ds(kk * IB, IB)], src_v)
            pltpu.sync_copy(dsts.at[r, s, pl.ds(kk * IB, IB)], dst_v)
            gather(0, rows0, sem0)
            gather(1, rows1, sem1)

            def pair(j, carry2):
              b0 = 2 * j
              gwait(rows0, sem0)
              pltpu.sync_copy(rows0, acc.at[dst_v.at[b0]], add=True)
              gather(jnp.minimum(b0 + 2, IB - 1), rows0, sem0)
              gwait(rows1, sem1)
              pltpu.sync_copy(rows1, acc.at[dst_v.at[b0 + 1]], add=True)
              gather(jnp.minimum(b0 + 3, IB - 1), rows1, sem1)
              return carry2
            lax.fori_loop(0, IB // 2, pair, 0)
            gwait(rows0, sem0)
            gwait(rows1, sem1)
            return carry
          lax.fori_loop(0, NB // IB, chunk, 0)
          plsc.subcore_barrier()
          dump(r, CW * q)
          plsc.subcore_barrier()
          return carry0
        lax.fori_loop(0, N_REL, rel_body, 0)

    if with_counts:
      one16 = jnp.ones((LANES,), jnp.float32)

      def fill_o(i, carry):
        rows0[i, pl.ds(0, LANES)] = one16
        rows0[i, pl.ds(LANES, LANES)] = one16
        return carry
      lax.fori_loop(0, K, fill_o, 0)

      def cnt_body(r, carry0):
        @pl.when(r % NC == c)
        def _():
          zero_acc()
          plsc.subcore_barrier()

          def chunk(kk, carry):
            pltpu.sync_copy(dsts.at[r, s, pl.ds(kk * IB, IB)], dst_v)

            def fire(b, carry2):
              pltpu.async_copy(rows0, acc.at[dst_v.at[b]], sem1, add=True)
              return carry2
            lax.fori_loop(0, IB, fire, 0)

            def drain(b, carry2):
              pltpu.make_async_copy(rows0, acc.at[dst_v.at[0]], sem1).wait()
              return carry2
            lax.fori_loop(0, IB, drain, 0)
            return carry
          lax.fori_loop(0, NB // IB, chunk, 0)
          plsc.subcore_barrier()

          @pl.when(s < NS - 1)
          def _():
            pltpu.sync_copy(acc.at[pl.ds(n0, RPT), pl.ds(0, 8)],
                            cnt_out.at[r, pl.ds(n0, RPT)])
          @pl.when(s == NS - 1)
          def _():
            last = (NS - 1) * RPT
            pltpu.sync_copy(acc.at[pl.ds(last, N_NODES - last), pl.ds(0, 8)],
                            cnt_out.at[r, pl.ds(last, N_NODES - last)])
          plsc.subcore_barrier()
        return carry0
      lax.fori_loop(0, N_REL, cnt_body, 0)

  return pl.kernel(body, out_type=out_type, mesh=mesh, scratch_types=scratch,
                   compiler_params=pltpu.CompilerParams(
                       use_tc_tiling_on_sc=False))


_BLK = 2000
_GRID = N_NODES // _BLK
_AROWS1 = _BLK * DIM // 128


def _tc_layer1(emb, a1p, cnt, w_root, w_rel_s, b1, gamma, beta):
  def body(emb_b, a_b, cnt_b, wr_b, ws_b, b1_b, g_b, be_b, h_o):
    x = emb_b[...]
    a = a_b[...]
    cnt = cnt_b[:, :, 0]
    parts = []
    for r in range(N_REL):
      d = jnp.clip(cnt[r], 1.0, None)[:, None]
      parts.append(a[r] / d)
    mean_all = jnp.concatenate(parts, axis=1)
    acc = (jnp.dot(x, wr_b[...], preferred_element_type=jnp.float32)
           + jnp.dot(mean_all, ws_b[...], preferred_element_type=jnp.float32)
           + b1_b[...])
    h = acc * (g_b[...] / np.sqrt(1.0 + EPS)) + be_b[...]
    h_o[...] = jnp.where(h > 0, h, jnp.exp(jnp.minimum(h, 0.0)) - 1.0)

  blk_n = lambda i: (i, 0)
  blk_rel = lambda i: (0, i, 0)
  blk_w = lambda i: (0, 0)
  return pl.pallas_call(
      body,
      grid=(_GRID,),
      in_specs=[
          pl.BlockSpec((_BLK, DIM), blk_n),
          pl.BlockSpec((N_REL, _BLK, DIM), blk_rel),
          pl.BlockSpec((N_REL, _BLK, 8), blk_rel),
          pl.BlockSpec((DIM, HIDDEN), blk_w),
          pl.BlockSpec((N_REL * DIM, HIDDEN), blk_w),
          pl.BlockSpec((1, HIDDEN), blk_w),
          pl.BlockSpec((1, HIDDEN), blk_w),
          pl.BlockSpec((1, HIDDEN), blk_w),
      ],
      out_specs=pl.BlockSpec((_BLK, HIDDEN), blk_n),
      out_shape=jax.ShapeDtypeStruct((N_NODES, HIDDEN), jnp.float32),
  )(emb, a1p, cnt, w_root, w_rel_s, b1, gamma, beta)


def _tc_layer2(h, a2, cnt, w_root, w_rel_s, b2):
  def body(h_b, a_b, cnt_b, wr_b, ws_b, b2_b, z_o):
    x = h_b[...]
    cnt = cnt_b[:, :, 0]
    parts = []
    for r in range(N_REL):
      d = jnp.clip(cnt[r], 1.0, None)[:, None]
      parts.append(a_b[r] / d)
    mean_all = jnp.concatenate(parts, axis=1)
    z_o[...] = (jnp.dot(x, wr_b[...], preferred_element_type=jnp.float32)
                + jnp.dot(mean_all, ws_b[...], preferred_element_type=jnp.float32)
                + b2_b[...])

  blk_n = lambda i: (i, 0)
  blk_rel = lambda i: (0, i, 0)
  blk_w = lambda i: (0, 0)
  return pl.pallas_call(
      body,
      grid=(_GRID,),
      in_specs=[
          pl.BlockSpec((_BLK, HIDDEN), blk_n),
          pl.BlockSpec((N_REL, _BLK, HIDDEN), blk_rel),
          pl.BlockSpec((N_REL, _BLK, 8), blk_rel),
          pl.BlockSpec((HIDDEN, DIM), blk_w),
          pl.BlockSpec((N_REL * HIDDEN, DIM), blk_w),
          pl.BlockSpec((1, DIM), blk_w),
      ],
      out_specs=pl.BlockSpec((_BLK, DIM), blk_n),
      out_shape=jax.ShapeDtypeStruct((N_NODES, DIM), jnp.float32),
  )(h, a2, cnt, w_root, w_rel_s, b2)


@jax.jit
def kernel(rel_edges_visible, emb, W_rel1, W_root1, b1, bn_gamma, bn_beta,
           W_rel2, W_root2, b2):
  edges = rel_edges_visible.astype(jnp.int32)
  src = edges[:, 0, :]
  dst = edges[:, 1, :]
  pad = E_PAD - E_PER_REL
  src_p = jnp.concatenate(
      [src, jnp.zeros((N_REL, pad), jnp.int32)], axis=1
  ).reshape(N_REL, NS, NB, K)
  dst_p = jnp.concatenate(
      [dst, jnp.full((N_REL, pad), TRASH, jnp.int32)], axis=1
  ).reshape(N_REL, NS, NB, K)

  emb_c0 = emb[:, 0:32]
  emb_c1 = emb[:, 32:64]

  agg1, cnt = _sc_agg_kernel(2, True)(src_p, dst_p, emb_c0, emb_c1)

  h = _tc_layer1(
      emb, agg1, cnt,
      W_root1, W_rel1.reshape(N_REL * DIM, HIDDEN),
      b1.reshape(1, HIDDEN), bn_gamma.reshape(1, HIDDEN),
      bn_beta.reshape(1, HIDDEN))

  h_tabs = [h[:, CW * q:CW * (q + 1)] for q in range(4)]
  (a2,) = _sc_agg_kernel(4, False)(src_p, dst_p, *h_tabs)

  z = _tc_layer2(h, a2, cnt,
                 W_root2, W_rel2.reshape(N_REL * HIDDEN, DIM),
                 b2.reshape(1, DIM))
  return z
